# Initial kernel scaffold; baseline (speedup 1.0000x reference)
import jax
import jax.numpy as jnp
from jax.experimental import pallas as pl


def kernel(input_, weight):
    raise NotImplementedError("write your pallas kernel here")


# speedup vs baseline: 1.0235x; 1.0235x over previous
import functools

import jax
import jax.numpy as jnp
from jax import lax
from jax.experimental import pallas as pl
from jax.experimental.pallas import tpu as pltpu
from jax.experimental.pallas import tpu_sc as plsc

V = 1000000
D = 32
B = 16384 * 50
NC, NS = 2, 16
NW = NC * NS
BPW = B // NW
CH = 128
NCH = BPW // CH

_mesh = plsc.VectorSubcoreMesh(core_axis_name="c", subcore_axis_name="s")


@functools.partial(
    pl.kernel,
    mesh=_mesh,
    out_type=jax.ShapeDtypeStruct((B, D), jnp.float32),
    scratch_types=[
        pltpu.VMEM((BPW,), jnp.int32),
        pltpu.VMEM((CH, D), jnp.float32),
        pltpu.SemaphoreType.DMA,
    ],
    compiler_params=pltpu.CompilerParams(use_tc_tiling_on_sc=False),
)
def _emb_lookup(table_hbm, idx_hbm, out_hbm, idx_v, rows_v, sem):
    wid = lax.axis_index("s") * NC + lax.axis_index("c")
    base = wid * BPW
    pltpu.sync_copy(idx_hbm.at[pl.ds(base, BPW)], idx_v)

    def chunk(j, carry):
        off = j * CH
        pltpu.async_copy(
            table_hbm.at[idx_v.at[pl.ds(off, CH)]], rows_v, sem
        ).wait()
        pltpu.sync_copy(rows_v, out_hbm.at[pl.ds(base + off, CH)])
        return carry

    lax.fori_loop(0, NCH, chunk, 0)


def kernel(input_, weight):
    idx = input_.reshape(-1).astype(jnp.int32)
    out = _emb_lookup(weight, idx)
    return out.reshape(input_.shape + (D,))


# speedup vs baseline: 1.3098x; 1.2797x over previous
import functools

import jax
import jax.numpy as jnp
from jax import lax
from jax.experimental import pallas as pl
from jax.experimental.pallas import tpu as pltpu
from jax.experimental.pallas import tpu_sc as plsc

V = 1000000
D = 32
B = 16384 * 50
NC, NS = 2, 16
NW = NC * NS
BPW = B // NW
CH = 128
NCH = BPW // CH
K = 10
NG = NCH // K
NPAIR = NG // 2

_mesh = plsc.VectorSubcoreMesh(core_axis_name="c", subcore_axis_name="s")


@functools.partial(
    pl.kernel,
    mesh=_mesh,
    out_type=jax.ShapeDtypeStruct((NW, NCH, CH, D), jnp.float32),
    scratch_types=[
        pltpu.VMEM((NCH, CH), jnp.int32),
        pltpu.VMEM((K, CH, D), jnp.float32),
        pltpu.VMEM((K, CH, D), jnp.float32),
        pltpu.SemaphoreType.DMA,
        pltpu.SemaphoreType.DMA,
        pltpu.SemaphoreType.DMA,
        pltpu.SemaphoreType.DMA,
    ],
    compiler_params=pltpu.CompilerParams(use_tc_tiling_on_sc=False),
)
def _emb_lookup(table_hbm, idx_hbm, out_hbm, idx_v, buf_a, buf_b,
                gsem_a, gsem_b, ssem_a, ssem_b):
    wid = lax.axis_index("s") * NC + lax.axis_index("c")
    pltpu.sync_copy(idx_hbm.at[wid], idx_v)

    bufs = (buf_a, buf_b)
    gsems = (gsem_a, gsem_b)
    ssems = (ssem_a, ssem_b)

    def start_gathers(bank, g0):
        for b in range(K):
            pltpu.async_copy(
                table_hbm.at[idx_v.at[g0 + b]], bufs[bank].at[b], gsems[bank]
            )

    def wait_gathers(bank, g0):
        pltpu.make_async_copy(
            out_hbm.at[wid, pl.ds(g0, K)], bufs[bank], gsems[bank]
        ).wait()

    def start_stores(bank, g0):
        pltpu.async_copy(bufs[bank], out_hbm.at[wid, pl.ds(g0, K)], ssems[bank])

    def wait_stores(bank, g0):
        pltpu.make_async_copy(
            bufs[bank], out_hbm.at[wid, pl.ds(g0, K)], ssems[bank]
        ).wait()

    start_gathers(0, 0)

    def body(t, carry):
        g_a = 2 * t * K
        g_b = g_a + K

        @pl.when(t > 0)
        def _():
            wait_stores(1, g_a - K)

        start_gathers(1, g_b)
        wait_gathers(0, g_a)
        start_stores(0, g_a)
        wait_gathers(1, g_b)
        wait_stores(0, g_a)

        @pl.when(t < NPAIR - 1)
        def _():
            start_gathers(0, g_a + 2 * K)

        start_stores(1, g_b)
        return carry

    lax.fori_loop(0, NPAIR, body, 0)
    wait_stores(1, NCH - K)


def kernel(input_, weight):
    idx = input_.reshape(NW, NCH, CH).astype(jnp.int32)
    out = _emb_lookup(weight, idx)
    return out.reshape(input_.shape + (D,))


# speedup vs baseline: 1.7988x; 1.3734x over previous
import functools

import jax
import jax.numpy as jnp
from jax import lax
from jax.experimental import pallas as pl
from jax.experimental.pallas import tpu as pltpu
from jax.experimental.pallas import tpu_sc as plsc

V = 1000000
D = 32
R = 16384
S = 50
NC, NS = 2, 16
NW = NC * NS
RPW = R // NW
G = 8
NGRP = RPW // G
NPAIR = NGRP // 2

_mesh = plsc.VectorSubcoreMesh(core_axis_name="c", subcore_axis_name="s")


@functools.partial(
    pl.kernel,
    mesh=_mesh,
    out_type=jax.ShapeDtypeStruct((R, S, D), jnp.float32),
    scratch_types=[
        pltpu.VMEM((RPW, S), jnp.int32),
        pltpu.VMEM((G, S, D), jnp.float32),
        pltpu.VMEM((G, S, D), jnp.float32),
        pltpu.SemaphoreType.DMA,
        pltpu.SemaphoreType.DMA,
        pltpu.SemaphoreType.DMA,
        pltpu.SemaphoreType.DMA,
    ],
    compiler_params=pltpu.CompilerParams(use_tc_tiling_on_sc=False),
)
def _emb_lookup(table_hbm, idx_hbm, out_hbm, idx_s, buf_a, buf_b,
                gsem_a, gsem_b, ssem_a, ssem_b):
    wid = lax.axis_index("s") * NC + lax.axis_index("c")
    wbase = wid * RPW
    pltpu.sync_copy(idx_hbm.at[pl.ds(wbase, RPW)], idx_s)

    bufs = (buf_a, buf_b)
    gsems = (gsem_a, gsem_b)
    ssems = (ssem_a, ssem_b)

    def start_gathers(bank, g0):
        for b in range(G):
            pltpu.async_copy(
                table_hbm.at[idx_s.at[g0 + b]], bufs[bank].at[b], gsems[bank]
            )

    def wait_gathers(bank, g0):
        pltpu.make_async_copy(
            out_hbm.at[pl.ds(wbase + g0, G)], bufs[bank], gsems[bank]
        ).wait()

    def start_stores(bank, g0):
        pltpu.async_copy(
            bufs[bank], out_hbm.at[pl.ds(wbase + g0, G)], ssems[bank]
        )

    def wait_stores(bank, g0):
        pltpu.make_async_copy(
            bufs[bank], out_hbm.at[pl.ds(wbase + g0, G)], ssems[bank]
        ).wait()

    start_gathers(0, 0)

    def body(t, carry):
        g_a = 2 * t * G
        g_b = g_a + G

        @pl.when(t > 0)
        def _():
            wait_stores(1, g_a - G)

        start_gathers(1, g_b)
        wait_gathers(0, g_a)
        start_stores(0, g_a)
        wait_gathers(1, g_b)
        wait_stores(0, g_a)

        @pl.when(t < NPAIR - 1)
        def _():
            start_gathers(0, g_a + 2 * G)

        start_stores(1, g_b)
        return carry

    lax.fori_loop(0, NPAIR, body, 0)
    wait_stores(1, RPW - G)


def kernel(input_, weight):
    return _emb_lookup(weight, input_.astype(jnp.int32))
